# baseline (speedup 1.0000x reference)
import functools

import jax
import jax.numpy as jnp
from jax import lax
from jax.experimental import pallas as pl
from jax.experimental.pallas import tpu as pltpu
from jax.experimental.pallas import tpu_sc as plsc

N_ROWS_TBL = 100000
N_EMB = 64
BATCH = 16384

NC = 2
NS = 16
NW = NC * NS
B_PER_W = BATCH // NW
N_CHUNK = B_PER_W // 128
TCB = 4096


def _tpose_tc(t_ref, c_ref, tb_ref, cb_ref, o_ref, otb_ref, ocb_ref):
    o_ref[:, 0:N_EMB] = t_ref[...].T
    o_ref[:, N_EMB:2 * N_EMB] = c_ref[...].T
    otb_ref[...] = tb_ref[0, :]
    ocb_ref[...] = cb_ref[0, :]


def _transpose_pack(teT, ceT, tbT, cbT):
    return pl.pallas_call(
        _tpose_tc,
        grid=((N_ROWS_TBL + TCB - 1) // TCB,),
        in_specs=[pl.BlockSpec((N_EMB, TCB), lambda i: (0, i)),
                  pl.BlockSpec((N_EMB, TCB), lambda i: (0, i)),
                  pl.BlockSpec((1, TCB), lambda i: (0, i)),
                  pl.BlockSpec((1, TCB), lambda i: (0, i))],
        out_specs=[pl.BlockSpec((TCB, 2 * N_EMB), lambda i: (i, 0)),
                   pl.BlockSpec((TCB,), lambda i: (i,)),
                   pl.BlockSpec((TCB,), lambda i: (i,))],
        out_shape=[jax.ShapeDtypeStruct((N_ROWS_TBL, 2 * N_EMB), jnp.float32),
                   jax.ShapeDtypeStruct((N_ROWS_TBL,), jnp.float32),
                   jax.ShapeDtypeStruct((N_ROWS_TBL,), jnp.float32)],
    )(teT, ceT, tbT, cbT)


def _mfnet_sc(task2d, cons2d, tece_hbm, tb_hbm, cb_hbm, wb_hbm, out_hbm,
              idx_t, idx_c, te_b, ce_b, tb_v, cb_v, wb_v, out_v, pbuf, sem):
    wid = lax.axis_index("s") * NC + lax.axis_index("c")
    base = wid * B_PER_W

    pltpu.sync_copy(task2d.at[pl.ds(wid * N_CHUNK, N_CHUNK)], idx_t)
    pltpu.sync_copy(cons2d.at[pl.ds(wid * N_CHUNK, N_CHUNK)], idx_c)
    pltpu.sync_copy(wb_hbm, wb_v)

    bias_copies = []
    for j in range(N_CHUNK):
        sl = pl.ds(j * 128, 128)
        bias_copies.append(pltpu.async_copy(tb_hbm.at[idx_t.at[j]], tb_v.at[sl], sem))
        bias_copies.append(pltpu.async_copy(cb_hbm.at[idx_c.at[j]], cb_v.at[sl], sem))

    def fire(j, buf_i):
        return (pltpu.async_copy(tece_hbm.at[idx_t.at[j]], te_b[buf_i], sem),
                pltpu.async_copy(tece_hbm.at[idx_c.at[j]], ce_b[buf_i], sem))

    inflight = fire(0, 0)

    wch = [wb_v[pl.ds(k * 16, 16)] for k in range(N_EMB // 16)]
    bd = wb_v[pl.ds(N_EMB, 16)][0]
    w_sc = [wch[k][l] for k in range(N_EMB // 16) for l in range(16)]
    s_tot = functools.reduce(lambda a, b: a + b, w_sc)

    for c in bias_copies:
        c.wait()

    lane = lax.iota(jnp.int32, 16)

    for j in range(N_CHUNK):
        cur = j % 2
        for c in inflight:
            c.wait()
        if j + 1 < N_CHUNK:
            inflight = fire(j + 1, 1 - cur)
        teb, ceb = te_b[cur], ce_b[cur]

        def group_body(g, _):
            r0 = g * 16
            for i in range(16):
                r = r0 + i
                p = teb[r, pl.ds(0, 16)] * ceb[r, pl.ds(N_EMB, 16)] * wch[0]
                for k in range(1, N_EMB // 16):
                    p = p + (teb[r, pl.ds(k * 16, 16)]
                             * ceb[r, pl.ds(N_EMB + k * 16, 16)] * wch[k])
                pbuf[pl.ds(i * 16, 16)] = p
            acc = plsc.load_gather(pbuf, [lane * 16])
            for c in range(1, 16):
                acc = acc + plsc.load_gather(pbuf, [lane * 16 + c])
            b0 = j * 128 + r0
            tb = tb_v[pl.ds(b0, 16)]
            cb = cb_v[pl.ds(b0, 16)]
            out_v[pl.ds(b0, 16)] = acc + s_tot * (tb + cb) + bd
            return 0

        lax.fori_loop(0, 128 // 16, group_body, 0)

    pltpu.sync_copy(out_v, out_hbm.at[pl.ds(base, B_PER_W)])


@jax.jit
def _mfnet(task2d, cons2d, tece, tb, cb, wb):
    mesh = plsc.VectorSubcoreMesh(core_axis_name="c", subcore_axis_name="s")
    f = functools.partial(
        pl.kernel,
        out_type=jax.ShapeDtypeStruct((BATCH,), jnp.float32),
        mesh=mesh,
        compiler_params=pltpu.CompilerParams(needs_layout_passes=False,
                                             use_tc_tiling_on_sc=False),
        scratch_types=[
            pltpu.VMEM((N_CHUNK, 128), jnp.int32),
            pltpu.VMEM((N_CHUNK, 128), jnp.int32),
            [pltpu.VMEM((128, 2 * N_EMB), jnp.float32)] * 2,
            [pltpu.VMEM((128, 2 * N_EMB), jnp.float32)] * 2,
            pltpu.VMEM((B_PER_W,), jnp.float32),
            pltpu.VMEM((B_PER_W,), jnp.float32),
            pltpu.VMEM((N_EMB + 16,), jnp.float32),
            pltpu.VMEM((B_PER_W,), jnp.float32),
            pltpu.VMEM((256,), jnp.float32),
            pltpu.SemaphoreType.DMA,
        ],
    )(_mfnet_sc)
    return f(task2d, cons2d, tece, tb, cb, wb)


def kernel(task, construct, task_emb, construct_emb, task_bias, construct_bias,
           W_dec, b_dec):
    task2d = task.astype(jnp.int32).reshape(BATCH // 128, 128)
    cons2d = construct.astype(jnp.int32).reshape(BATCH // 128, 128)
    tece, tb, cb = _transpose_pack(task_emb.T, construct_emb.T,
                                   task_bias.T, construct_bias.T)
    wb = jnp.concatenate([W_dec.reshape(N_EMB),
                          jnp.broadcast_to(b_dec, (16,)).astype(jnp.float32)])
    y = _mfnet(task2d, cons2d, tece, tb, cb, wb)
    return y.reshape(BATCH, 1)


# speedup vs baseline: 1.0750x; 1.0750x over previous
import functools

import jax
import jax.numpy as jnp
from jax import lax
from jax.experimental import pallas as pl
from jax.experimental.pallas import tpu as pltpu
from jax.experimental.pallas import tpu_sc as plsc

N_ROWS_TBL = 100000
N_EMB = 64
BATCH = 16384

NC = 2
NS = 16
NW = NC * NS
B_PER_W = BATCH // NW
N_CHUNK = B_PER_W // 128
TCB = 8192


def _tpose_tc(t_ref, c_ref, tb_ref, cb_ref, o_ref, otb_ref, ocb_ref):
    o_ref[:, 0:N_EMB] = t_ref[...].T
    o_ref[:, N_EMB:2 * N_EMB] = c_ref[...].T
    otb_ref[...] = tb_ref[0, :]
    ocb_ref[...] = cb_ref[0, :]


def _transpose_pack(teT, ceT, tbT, cbT):
    return pl.pallas_call(
        _tpose_tc,
        grid=((N_ROWS_TBL + TCB - 1) // TCB,),
        in_specs=[pl.BlockSpec((N_EMB, TCB), lambda i: (0, i)),
                  pl.BlockSpec((N_EMB, TCB), lambda i: (0, i)),
                  pl.BlockSpec((1, TCB), lambda i: (0, i)),
                  pl.BlockSpec((1, TCB), lambda i: (0, i))],
        out_specs=[pl.BlockSpec((TCB, 2 * N_EMB), lambda i: (i, 0)),
                   pl.BlockSpec((TCB,), lambda i: (i,)),
                   pl.BlockSpec((TCB,), lambda i: (i,))],
        out_shape=[jax.ShapeDtypeStruct((N_ROWS_TBL, 2 * N_EMB), jnp.float32),
                   jax.ShapeDtypeStruct((N_ROWS_TBL,), jnp.float32),
                   jax.ShapeDtypeStruct((N_ROWS_TBL,), jnp.float32)],
    )(teT, ceT, tbT, cbT)


def _mfnet_sc(task2d, cons2d, tece_hbm, tb_hbm, cb_hbm, wb_hbm, out_hbm,
              idx_t, idx_c, idx_t2, idx_c2, te_b, ce_b, tb_v, cb_v, wb_v,
              out_v, pbuf, sem):
    wid = lax.axis_index("s") * NC + lax.axis_index("c")
    base = wid * B_PER_W

    pltpu.sync_copy(task2d.at[pl.ds(wid * N_CHUNK, N_CHUNK)], idx_t)
    pltpu.sync_copy(cons2d.at[pl.ds(wid * N_CHUNK, N_CHUNK)], idx_c)
    pltpu.sync_copy(wb_hbm, wb_v)

    for j in range(N_CHUNK):
        for g in range(8):
            sl16 = pl.ds(g * 16, 16)
            idx_t2[j, sl16] = idx_t[j, sl16] * 2
            idx_c2[j, sl16] = idx_c[j, sl16] * 2 + 1

    bias_copies = []
    for j in range(N_CHUNK):
        sl = pl.ds(j * 128, 128)
        bias_copies.append(pltpu.async_copy(tb_hbm.at[idx_t.at[j]], tb_v.at[sl], sem))
        bias_copies.append(pltpu.async_copy(cb_hbm.at[idx_c.at[j]], cb_v.at[sl], sem))

    def fire(j, buf_i):
        return (pltpu.async_copy(tece_hbm.at[idx_t2.at[j]], te_b[buf_i], sem),
                pltpu.async_copy(tece_hbm.at[idx_c2.at[j]], ce_b[buf_i], sem))

    inflight = fire(0, 0)

    wch = [wb_v[pl.ds(k * 16, 16)] for k in range(N_EMB // 16)]
    bd = wb_v[pl.ds(N_EMB, 16)][0]
    w_sc = [wch[k][l] for k in range(N_EMB // 16) for l in range(16)]
    s_tot = functools.reduce(lambda a, b: a + b, w_sc)

    for c in bias_copies:
        c.wait()

    lane = lax.iota(jnp.int32, 16)

    for j in range(N_CHUNK):
        cur = j % 2
        for c in inflight:
            c.wait()
        if j + 1 < N_CHUNK:
            inflight = fire(j + 1, 1 - cur)
        teb, ceb = te_b[cur], ce_b[cur]

        def group_body(g, _):
            r0 = g * 16
            for i in range(16):
                r = r0 + i
                p = teb[r, pl.ds(0, 16)] * ceb[r, pl.ds(0, 16)] * wch[0]
                for k in range(1, N_EMB // 16):
                    p = p + (teb[r, pl.ds(k * 16, 16)]
                             * ceb[r, pl.ds(k * 16, 16)] * wch[k])
                pbuf[pl.ds(i * 16, 16)] = p
            acc = plsc.load_gather(pbuf, [lane * 16])
            for c in range(1, 16):
                acc = acc + plsc.load_gather(pbuf, [lane * 16 + c])
            b0 = j * 128 + r0
            tb = tb_v[pl.ds(b0, 16)]
            cb = cb_v[pl.ds(b0, 16)]
            out_v[pl.ds(b0, 16)] = acc + s_tot * (tb + cb) + bd
            return 0

        lax.fori_loop(0, 128 // 16, group_body, 0)

    pltpu.sync_copy(out_v, out_hbm.at[pl.ds(base, B_PER_W)])


@jax.jit
def _mfnet(task2d, cons2d, tece, tb, cb, wb):
    mesh = plsc.VectorSubcoreMesh(core_axis_name="c", subcore_axis_name="s")
    f = functools.partial(
        pl.kernel,
        out_type=jax.ShapeDtypeStruct((BATCH,), jnp.float32),
        mesh=mesh,
        compiler_params=pltpu.CompilerParams(needs_layout_passes=False,
                                             use_tc_tiling_on_sc=False),
        scratch_types=[
            pltpu.VMEM((N_CHUNK, 128), jnp.int32),
            pltpu.VMEM((N_CHUNK, 128), jnp.int32),
            pltpu.VMEM((N_CHUNK, 128), jnp.int32),
            pltpu.VMEM((N_CHUNK, 128), jnp.int32),
            [pltpu.VMEM((128, N_EMB), jnp.float32)] * 2,
            [pltpu.VMEM((128, N_EMB), jnp.float32)] * 2,
            pltpu.VMEM((B_PER_W,), jnp.float32),
            pltpu.VMEM((B_PER_W,), jnp.float32),
            pltpu.VMEM((N_EMB + 16,), jnp.float32),
            pltpu.VMEM((B_PER_W,), jnp.float32),
            pltpu.VMEM((256,), jnp.float32),
            pltpu.SemaphoreType.DMA,
        ],
    )(_mfnet_sc)
    return f(task2d, cons2d, tece, tb, cb, wb)


def kernel(task, construct, task_emb, construct_emb, task_bias, construct_bias,
           W_dec, b_dec):
    task2d = task.astype(jnp.int32).reshape(BATCH // 128, 128)
    cons2d = construct.astype(jnp.int32).reshape(BATCH // 128, 128)
    tece, tb, cb = _transpose_pack(task_emb.T, construct_emb.T,
                                   task_bias.T, construct_bias.T)
    tece2 = tece.reshape(2 * N_ROWS_TBL, N_EMB)
    wb = jnp.concatenate([W_dec.reshape(N_EMB),
                          jnp.broadcast_to(b_dec, (16,)).astype(jnp.float32)])
    y = _mfnet(task2d, cons2d, tece2, tb, cb, wb)
    return y.reshape(BATCH, 1)
